# baseline (speedup 1.0000x reference)
import functools

import jax
import jax.numpy as jnp
from jax import lax
from jax.experimental import pallas as pl
from jax.experimental.pallas import tpu as pltpu
from jax.experimental.pallas import tpu_sc as plsc

_NC, _NS = 2, 16
_NW = _NC * _NS
_CHUNK = 128


def _vq_body(x_ref, w_ref, idx_ref, loss_ref, w2r_ref, *, C, IPB):
    i = pl.program_id(0)
    w = w_ref[...]

    xr = jnp.concatenate([x_ref[j].T for j in range(IPB)], axis=0)

    @pl.when(i == 0)
    def _():
        w2 = jnp.sum(w * w, axis=1, keepdims=True)
        w2r_ref[...] = w2.T

    x2 = jnp.sum(xr * xr, axis=1, keepdims=True)
    s2 = lax.dot_general(xr, w + w, (((1,), (1,)), ((), ())),
                         preferred_element_type=jnp.float32)
    d2 = jnp.maximum(x2 - s2 + w2r_ref[...], 0.0)

    dist = jnp.sqrt(d2)

    minv = jnp.min(dist, axis=1, keepdims=True)
    iota = lax.broadcasted_iota(jnp.int32, dist.shape, 1)
    idx = jnp.min(jnp.where(dist == minv, iota, C), axis=1,
                  keepdims=True)
    idx_ref[0] = idx

    part = jnp.sum(jnp.min(d2, axis=1))

    @pl.when(i == 0)
    def _():
        loss_ref[0, 0] = 0.0

    loss_ref[0, 0] += part


def _codebook_gather(table, idx2d, B, D):
    bpw = B // _NW
    k = bpw // _CHUNK
    mesh = plsc.VectorSubcoreMesh(core_axis_name="c", subcore_axis_name="s")

    @functools.partial(
        pl.kernel, mesh=mesh,
        out_type=jax.ShapeDtypeStruct((B, D), jnp.float32),
        scratch_types=[
            pltpu.VMEM((k, _CHUNK), jnp.int32),
            pltpu.VMEM((bpw, D), jnp.float32),
            pltpu.SemaphoreType.DMA,
        ],
    )
    def gather_k(table_hbm, idx_hbm, out_hbm, idx_v, rows_v, sem):
        wid = lax.axis_index("s") * _NC + lax.axis_index("c")
        pltpu.sync_copy(idx_hbm.at[pl.ds(wid * k, k)], idx_v)
        copies = [
            pltpu.async_copy(table_hbm.at[idx_v.at[j]],
                             rows_v.at[pl.ds(j * _CHUNK, _CHUNK)], sem)
            for j in range(k)
        ]
        for c in copies:
            c.wait()
        pltpu.sync_copy(rows_v, out_hbm.at[pl.ds(wid * bpw, bpw)])

    return gather_k(table, idx2d)


def _tr_body(q_ref, o_ref, *, D):
    o_ref[0] = q_ref[0][:, :D].T


def _to_channel_major(qf, N, D, P):
    q3 = qf.reshape(N, P, 128)
    return pl.pallas_call(
        functools.partial(_tr_body, D=D),
        grid=(N,),
        in_specs=[pl.BlockSpec((1, P, 128), lambda i: (i, 0, 0))],
        out_specs=pl.BlockSpec((1, D, P), lambda i: (i, 0, 0)),
        out_shape=jax.ShapeDtypeStruct((N, D, P), jnp.float32),
    )(q3)


def kernel(inputs, embeddings_weight):
    N, D, H, W = inputs.shape
    C = embeddings_weight.shape[0]
    P = H * W
    B = N * P
    x3 = inputs.reshape(N, D, P)

    IPB = 2
    PB = IPB * P
    idx3, loss_sum = pl.pallas_call(
        functools.partial(_vq_body, C=C, IPB=IPB),
        grid=(N // IPB,),
        in_specs=[
            pl.BlockSpec((IPB, D, P), lambda i: (i, 0, 0)),
            pl.BlockSpec((C, D), lambda i: (0, 0)),
        ],
        out_specs=[
            pl.BlockSpec((1, PB, 1), lambda i: (i, 0, 0)),
            pl.BlockSpec((1, 1), lambda i: (0, 0), memory_space=pltpu.SMEM),
        ],
        out_shape=[
            jax.ShapeDtypeStruct((N // IPB, PB, 1), jnp.int32),
            jax.ShapeDtypeStruct((1, 1), jnp.float32),
        ],
        scratch_shapes=[
            pltpu.VMEM((1, C), jnp.float32),
        ],
    )(x3, embeddings_weight)

    idx2d = idx3.reshape(B // _CHUNK, _CHUNK)
    table_pad = jnp.pad(embeddings_weight, ((0, 0), (0, 128 - D)))
    qf = _codebook_gather(table_pad, idx2d, B, 128)
    quantized_st = _to_channel_major(qf, N, D, P).reshape(N, D, H, W)
    c_loss = loss_sum[0, 0] * jnp.float32(1.25) / jnp.float32(B * D)
    return (c_loss, quantized_st)


# speedup vs baseline: 1.2675x; 1.2675x over previous
import functools

import jax
import jax.numpy as jnp
from jax import lax
from jax.experimental import pallas as pl
from jax.experimental.pallas import tpu as pltpu
from jax.experimental.pallas import tpu_sc as plsc

_NC, _NS = 2, 16
_NW = _NC * _NS
_CHUNK = 128


def _vq_body(x_ref, w_ref, idx_ref, loss_ref, w2r_ref, *, C, IPB):
    i = pl.program_id(0)
    w = w_ref[...]

    xr = jnp.concatenate([x_ref[j].T for j in range(IPB)], axis=0)

    @pl.when(i == 0)
    def _():
        w2 = jnp.sum(w * w, axis=1, keepdims=True)
        w2r_ref[...] = w2.T

    x2 = jnp.sum(xr * xr, axis=1, keepdims=True)
    s2 = lax.dot_general(xr, w + w, (((1,), (1,)), ((), ())),
                         preferred_element_type=jnp.float32)
    d2 = jnp.maximum(x2 - s2 + w2r_ref[...], 0.0)

    dist = jnp.sqrt(d2)

    minv = jnp.min(dist, axis=1, keepdims=True)
    iota = lax.broadcasted_iota(jnp.int32, dist.shape, 1)
    idx = jnp.min(jnp.where(dist == minv, iota, C), axis=1,
                  keepdims=True)
    idx_ref[0] = idx

    part = jnp.sum(minv * minv)

    @pl.when(i == 0)
    def _():
        loss_ref[0, 0] = 0.0

    loss_ref[0, 0] += part


def _codebook_gather(table, idx2d, B, D):
    bpw = B // _NW
    k = bpw // _CHUNK
    mesh = plsc.VectorSubcoreMesh(core_axis_name="c", subcore_axis_name="s")

    @functools.partial(
        pl.kernel, mesh=mesh,
        out_type=jax.ShapeDtypeStruct((B, D), jnp.float32),
        scratch_types=[
            pltpu.VMEM((k, _CHUNK), jnp.int32),
            pltpu.VMEM((bpw, D), jnp.float32),
            pltpu.SemaphoreType.DMA,
        ],
    )
    def gather_k(table_hbm, idx_hbm, out_hbm, idx_v, rows_v, sem):
        wid = lax.axis_index("s") * _NC + lax.axis_index("c")
        pltpu.sync_copy(idx_hbm.at[pl.ds(wid * k, k)], idx_v)
        copies = [
            pltpu.async_copy(table_hbm.at[idx_v.at[j]],
                             rows_v.at[pl.ds(j * _CHUNK, _CHUNK)], sem)
            for j in range(k)
        ]
        for c in copies:
            c.wait()
        pltpu.sync_copy(rows_v, out_hbm.at[pl.ds(wid * bpw, bpw)])

    return gather_k(table, idx2d)


def _tr_body(q_ref, o_ref, *, D):
    o_ref[0] = q_ref[0][:, :D].T


def _to_channel_major(qf, N, D, P):
    q3 = qf.reshape(N, P, 128)
    return pl.pallas_call(
        functools.partial(_tr_body, D=D),
        grid=(N,),
        in_specs=[pl.BlockSpec((1, P, 128), lambda i: (i, 0, 0))],
        out_specs=pl.BlockSpec((1, D, P), lambda i: (i, 0, 0)),
        out_shape=jax.ShapeDtypeStruct((N, D, P), jnp.float32),
    )(q3)


def kernel(inputs, embeddings_weight):
    N, D, H, W = inputs.shape
    C = embeddings_weight.shape[0]
    P = H * W
    B = N * P
    x3 = inputs.reshape(N, D, P)

    IPB = 4
    PB = IPB * P
    idx3, loss_sum = pl.pallas_call(
        functools.partial(_vq_body, C=C, IPB=IPB),
        grid=(N // IPB,),
        in_specs=[
            pl.BlockSpec((IPB, D, P), lambda i: (i, 0, 0)),
            pl.BlockSpec((C, D), lambda i: (0, 0)),
        ],
        out_specs=[
            pl.BlockSpec((1, PB, 1), lambda i: (i, 0, 0)),
            pl.BlockSpec((1, 1), lambda i: (0, 0), memory_space=pltpu.SMEM),
        ],
        out_shape=[
            jax.ShapeDtypeStruct((N // IPB, PB, 1), jnp.int32),
            jax.ShapeDtypeStruct((1, 1), jnp.float32),
        ],
        scratch_shapes=[
            pltpu.VMEM((1, C), jnp.float32),
        ],
    )(x3, embeddings_weight)

    idx2d = idx3.reshape(B // _CHUNK, _CHUNK)
    table_pad = jnp.pad(embeddings_weight, ((0, 0), (0, 128 - D)))
    qf = _codebook_gather(table_pad, idx2d, B, 128)
    quantized_st = jnp.transpose(
        qf.reshape(N, P, 128)[:, :, :D], (0, 2, 1)).reshape(N, D, H, W)
    c_loss = loss_sum[0, 0] * jnp.float32(1.25) / jnp.float32(B * D)
    return (c_loss, quantized_st)


# speedup vs baseline: 1.2677x; 1.0001x over previous
import functools

import jax
import jax.numpy as jnp
from jax import lax
from jax.experimental import pallas as pl
from jax.experimental.pallas import tpu as pltpu
from jax.experimental.pallas import tpu_sc as plsc

_NC, _NS = 2, 16
_NW = _NC * _NS
_CHUNK = 128


def _vq_body(x_ref, w_ref, idx_ref, loss_ref, w2r_ref, *, C, IPB):
    i = pl.program_id(0)
    w = w_ref[...]

    xr = jnp.concatenate([x_ref[j].T for j in range(IPB)], axis=0)

    @pl.when(i == 0)
    def _():
        w2 = jnp.sum(w * w, axis=1, keepdims=True)
        w2r_ref[...] = w2.T

    x2 = jnp.sum(xr * xr, axis=1, keepdims=True)
    s2 = lax.dot_general(xr, w + w, (((1,), (1,)), ((), ())),
                         preferred_element_type=jnp.float32)
    d2 = jnp.maximum(x2 - s2 + w2r_ref[...], 0.0)

    dist = jnp.sqrt(d2)

    minv = jnp.min(dist, axis=1, keepdims=True)
    iota = lax.broadcasted_iota(jnp.int32, dist.shape, 1)
    idx = jnp.min(jnp.where(dist == minv, iota, C), axis=1,
                  keepdims=True)
    idx_ref[0] = idx

    part = jnp.sum(minv * minv)

    @pl.when(i == 0)
    def _():
        loss_ref[0, 0] = 0.0

    loss_ref[0, 0] += part


def _codebook_gather(table, idx2d, B, D):
    bpw = B // _NW
    k = bpw // _CHUNK
    mesh = plsc.VectorSubcoreMesh(core_axis_name="c", subcore_axis_name="s")

    @functools.partial(
        pl.kernel, mesh=mesh,
        out_type=jax.ShapeDtypeStruct((B, D), jnp.float32),
        scratch_types=[
            pltpu.VMEM((k, _CHUNK), jnp.int32),
            pltpu.VMEM((bpw, D), jnp.float32),
            pltpu.SemaphoreType.DMA,
        ],
    )
    def gather_k(table_hbm, idx_hbm, out_hbm, idx_v, rows_v, sem):
        wid = lax.axis_index("s") * _NC + lax.axis_index("c")
        pltpu.sync_copy(idx_hbm.at[pl.ds(wid * k, k)], idx_v)
        copies = [
            pltpu.async_copy(table_hbm.at[idx_v.at[j]],
                             rows_v.at[pl.ds(j * _CHUNK, _CHUNK)], sem)
            for j in range(k)
        ]
        for c in copies:
            c.wait()
        pltpu.sync_copy(rows_v, out_hbm.at[pl.ds(wid * bpw, bpw)])

    return gather_k(table, idx2d)


def kernel(inputs, embeddings_weight):
    N, D, H, W = inputs.shape
    C = embeddings_weight.shape[0]
    P = H * W
    B = N * P
    x3 = inputs.reshape(N, D, P)

    IPB = 4
    PB = IPB * P
    idx3, loss_sum = pl.pallas_call(
        functools.partial(_vq_body, C=C, IPB=IPB),
        grid=(N // IPB,),
        in_specs=[
            pl.BlockSpec((IPB, D, P), lambda i: (i, 0, 0)),
            pl.BlockSpec((C, D), lambda i: (0, 0)),
        ],
        out_specs=[
            pl.BlockSpec((1, PB, 1), lambda i: (i, 0, 0)),
            pl.BlockSpec((1, 1), lambda i: (0, 0), memory_space=pltpu.SMEM),
        ],
        out_shape=[
            jax.ShapeDtypeStruct((N // IPB, PB, 1), jnp.int32),
            jax.ShapeDtypeStruct((1, 1), jnp.float32),
        ],
        scratch_shapes=[
            pltpu.VMEM((1, C), jnp.float32),
        ],
    )(x3, embeddings_weight)

    idx2d = idx3.reshape(B // _CHUNK, _CHUNK)
    table_pad = jnp.pad(embeddings_weight, ((0, 0), (0, 128 - D)))
    qf = _codebook_gather(table_pad, idx2d, B, 128)
    quantized_st = jnp.transpose(
        qf.reshape(N, P, 128)[:, :, :D], (0, 2, 1)).reshape(N, D, H, W)
    c_loss = loss_sum[0, 0] * jnp.float32(1.25) / jnp.float32(B * D)
    return (c_loss, quantized_st)
